# initial kernel scaffold (unmeasured)
import jax
import jax.numpy as jnp
from jax import lax
from jax.experimental import pallas as pl
from jax.experimental.pallas import tpu as pltpu

M = 4096
N = 8192
K_LOC = 4096
HALF = M // 2

M_BLK = 1024
N_BLK = 2048
K_BLK = 512


def _mm_body(x_ref, dy_ref, o_ref):
    k = pl.program_id(2)

    @pl.when(k == 0)
    def _():
        o_ref[...] = jnp.zeros_like(o_ref)

    xb = x_ref[...].astype(jnp.bfloat16)
    db = dy_ref[...].astype(jnp.bfloat16)
    o_ref[...] += lax.dot_general(
        xb, db, (((0,), (0,)), ((), ())),
        preferred_element_type=jnp.float32,
    )


def _matmul(x, dy):
    grid = (M // M_BLK, N // N_BLK, K_LOC // K_BLK)
    return pl.pallas_call(
        _mm_body,
        grid=grid,
        in_specs=[
            pl.BlockSpec((K_BLK, M_BLK), lambda i, j, k: (k, i)),
            pl.BlockSpec((K_BLK, N_BLK), lambda i, j, k: (k, j)),
        ],
        out_specs=pl.BlockSpec((M_BLK, N_BLK), lambda i, j, k: (i, j)),
        out_shape=jax.ShapeDtypeStruct((M, N), jnp.float32),
        compiler_params=pltpu.CompilerParams(
            dimension_semantics=("parallel", "parallel", "arbitrary"),
        ),
    )(x, dy)


def _exchange_body(p_ref, out_ref, send_sem, recv_sem):
    my_x = lax.axis_index("x")
    my_y = lax.axis_index("y")
    my_z = lax.axis_index("z")
    other = 1 - my_y
    rdma = pltpu.make_async_remote_copy(
        src_ref=p_ref.at[pl.ds(other * HALF, HALF), :],
        dst_ref=out_ref,
        send_sem=send_sem,
        recv_sem=recv_sem,
        device_id=(my_x, other, my_z),
        device_id_type=pl.DeviceIdType.MESH,
    )
    rdma.start()
    rdma.wait()


def _exchange(p):
    return pl.pallas_call(
        _exchange_body,
        out_shape=jax.ShapeDtypeStruct((HALF, N), jnp.float32),
        in_specs=[pl.BlockSpec(memory_space=pltpu.ANY)],
        out_specs=pl.BlockSpec(memory_space=pltpu.ANY),
        scratch_shapes=[pltpu.SemaphoreType.DMA, pltpu.SemaphoreType.DMA],
    )(p)


def _add_body(a_ref, b_ref, o_ref):
    o_ref[...] = a_ref[...] + b_ref[...]


_ADD_BLK = 256


def _add(a, b):
    grid = (HALF // _ADD_BLK,)
    spec = pl.BlockSpec((_ADD_BLK, N), lambda i: (i, 0))
    return pl.pallas_call(
        _add_body,
        grid=grid,
        in_specs=[spec, spec],
        out_specs=spec,
        out_shape=jax.ShapeDtypeStruct((HALF, N), jnp.float32),
    )(a, b)


def kernel(x, dy):
    my_y = lax.axis_index("y")
    p = _matmul(x, dy)
    recv = _exchange(p)
    mine = lax.dynamic_slice(p, (my_y * HALF, 0), (HALF, N))
    return _add(mine, recv)


# baseline (device time: 1232578 ns/iter reference)
import jax
import jax.numpy as jnp
from jax import lax
from jax.experimental import pallas as pl
from jax.experimental.pallas import tpu as pltpu

M = 4096
N = 8192
K_LOC = 4096
HALF = M // 2

M_BLK = 1024
N_BLK = 2048
K_BLK = 512


def _mm_body(x_ref, dy_ref, o_ref):
    k = pl.program_id(2)

    @pl.when(k == 0)
    def _():
        o_ref[...] = jnp.zeros_like(o_ref)

    xb = x_ref[...].astype(jnp.bfloat16)
    db = dy_ref[...].astype(jnp.bfloat16)
    o_ref[...] += lax.dot_general(
        xb, db, (((0,), (0,)), ((), ())),
        preferred_element_type=jnp.float32,
    )


def _matmul(x, dy):
    grid = (M // M_BLK, N // N_BLK, K_LOC // K_BLK)
    return pl.pallas_call(
        _mm_body,
        grid=grid,
        in_specs=[
            pl.BlockSpec((K_BLK, M_BLK), lambda i, j, k: (k, i)),
            pl.BlockSpec((K_BLK, N_BLK), lambda i, j, k: (k, j)),
        ],
        out_specs=pl.BlockSpec((M_BLK, N_BLK), lambda i, j, k: (i, j)),
        out_shape=jax.ShapeDtypeStruct((M, N), jnp.float32),
        compiler_params=pltpu.CompilerParams(
            dimension_semantics=("parallel", "parallel", "arbitrary"),
        ),
    )(x, dy)


def _exchange_body(p_ref, out_ref, send_sem, recv_sem):
    my_x = lax.axis_index("x")
    my_y = lax.axis_index("y")
    my_z = lax.axis_index("z")
    other = 1 - my_y
    rdma = pltpu.make_async_remote_copy(
        src_ref=p_ref.at[pl.ds(other * HALF, HALF), :],
        dst_ref=out_ref,
        send_sem=send_sem,
        recv_sem=recv_sem,
        device_id=(my_x, other, my_z),
        device_id_type=pl.DeviceIdType.MESH,
    )
    rdma.start()
    rdma.wait()


def _exchange(p):
    return pl.pallas_call(
        _exchange_body,
        out_shape=jax.ShapeDtypeStruct((HALF, N), jnp.float32),
        in_specs=[pl.BlockSpec(memory_space=pl.ANY)],
        out_specs=pl.BlockSpec(memory_space=pl.ANY),
        scratch_shapes=[pltpu.SemaphoreType.DMA, pltpu.SemaphoreType.DMA],
    )(p)


def _add_body(a_ref, b_ref, o_ref):
    o_ref[...] = a_ref[...] + b_ref[...]


_ADD_BLK = 128


def _add(a, b):
    grid = (HALF // _ADD_BLK,)
    spec = pl.BlockSpec((_ADD_BLK, N), lambda i: (i, 0))
    return pl.pallas_call(
        _add_body,
        grid=grid,
        in_specs=[spec, spec],
        out_specs=spec,
        out_shape=jax.ShapeDtypeStruct((HALF, N), jnp.float32),
    )(a, b)


def kernel(x, dy):
    my_y = lax.axis_index("y")
    p = _matmul(x, dy)
    recv = _exchange(p)
    mine = lax.dynamic_slice(p, (my_y * HALF, 0), (HALF, N))
    return _add(mine, recv)


# device time: 596693 ns/iter; 2.0657x vs baseline; 2.0657x over previous
import jax
import jax.numpy as jnp
from jax import lax
from jax.experimental import pallas as pl
from jax.experimental.pallas import tpu as pltpu

M = 4096
N = 8192
K_LOC = 4096
HALF = M // 2
QTR = HALF // 4

M_BLK = 2048
N_BLK = 2048
K_BLK = 512


def _mm_body(x_ref, dy_ref, o_ref, acc_ref):
    k = pl.program_id(2)

    @pl.when(k == 0)
    def _():
        acc_ref[...] = jnp.zeros_like(acc_ref)

    xb = x_ref[...].astype(jnp.bfloat16)
    db = dy_ref[...].astype(jnp.bfloat16)
    acc_ref[...] += lax.dot_general(
        xb, db, (((0,), (0,)), ((), ())),
        preferred_element_type=jnp.float32,
    )

    @pl.when(k == K_LOC // K_BLK - 1)
    def _():
        o_ref[...] = acc_ref[...].astype(jnp.bfloat16)


def _matmul(x, dy):
    grid = (M // M_BLK, N // N_BLK, K_LOC // K_BLK)
    return pl.pallas_call(
        _mm_body,
        grid=grid,
        in_specs=[
            pl.BlockSpec((K_BLK, M_BLK), lambda i, j, k: (k, i)),
            pl.BlockSpec((K_BLK, N_BLK), lambda i, j, k: (k, j)),
        ],
        out_specs=pl.BlockSpec((M_BLK, N_BLK), lambda i, j, k: (i, j)),
        out_shape=jax.ShapeDtypeStruct((M, N), jnp.bfloat16),
        scratch_shapes=[pltpu.VMEM((M_BLK, N_BLK), jnp.float32)],
        compiler_params=pltpu.CompilerParams(
            dimension_semantics=("parallel", "parallel", "arbitrary"),
            vmem_limit_bytes=100 * 1024 * 1024,
        ),
    )(x, dy)


def _comm_body(p_ref, out_ref, send_sems, recv_sems):
    my_x = lax.axis_index("x")
    my_y = lax.axis_index("y")
    my_z = lax.axis_index("z")
    other = 1 - my_y

    r = 2 * my_x + (my_z ^ my_x)
    opp = (r + 2) % 4
    left_r = (r + 3) % 4
    right_r = (r + 1) % 4
    r_even = (r % 2) == 0
    right_x = jnp.where(r_even, my_x, 1 - my_x)
    right_z = jnp.where(r_even, 1 - my_z, my_z)
    left_x = jnp.where(r_even, 1 - my_x, my_x)
    left_z = jnp.where(r_even, my_z, 1 - my_z)

    def slot(s):
        return out_ref.at[pl.ds(s * QTR, QTR), :]

    def psrc(s):
        return p_ref.at[pl.ds(other * HALF + s * QTR, QTR), :]

    def rdma(src, dst, i, dev):
        return pltpu.make_async_remote_copy(
            src_ref=src,
            dst_ref=dst,
            send_sem=send_sems.at[i],
            recv_sem=recv_sems.at[i],
            device_id=dev,
            device_id_type=pl.DeviceIdType.MESH,
        )

    y_nbr = (my_x, other, my_z)
    right = (right_x, my_y, right_z)
    left = (left_x, my_y, left_z)

    a1 = rdma(psrc(r), slot(r), 0, y_nbr)
    a2 = rdma(psrc(opp), slot(opp), 1, y_nbr)
    a1.start()
    a2.start()

    a1.wait_recv()
    b_right = rdma(slot(r), slot(r), 2, right)
    b_left = rdma(slot(r), slot(r), 3, left)
    b_right.start()
    b_left.start()

    a2.wait_recv()
    rdma(psrc(left_r), slot(left_r), 2, left).wait_recv()
    rdma(psrc(right_r), slot(right_r), 3, right).wait_recv()

    a1.wait_send()
    a2.wait_send()
    b_right.wait_send()
    b_left.wait_send()


def _comm(p):
    return pl.pallas_call(
        _comm_body,
        out_shape=jax.ShapeDtypeStruct((HALF, N), jnp.bfloat16),
        in_specs=[pl.BlockSpec(memory_space=pl.ANY)],
        out_specs=pl.BlockSpec(memory_space=pl.ANY),
        scratch_shapes=[
            pltpu.SemaphoreType.DMA((4,)),
            pltpu.SemaphoreType.DMA((4,)),
        ],
    )(p)


def _add_body(a_ref, b_ref, o_ref):
    o_ref[...] = a_ref[...].astype(jnp.float32) + b_ref[...].astype(jnp.float32)


_ADD_BLK = 256


def _add(a, b):
    grid = (HALF // _ADD_BLK,)
    spec = pl.BlockSpec((_ADD_BLK, N), lambda i: (i, 0))
    return pl.pallas_call(
        _add_body,
        grid=grid,
        in_specs=[spec, spec],
        out_specs=pl.BlockSpec((_ADD_BLK, N), lambda i: (i, 0)),
        out_shape=jax.ShapeDtypeStruct((HALF, N), jnp.float32),
    )(a, b)


def kernel(x, dy):
    my_y = lax.axis_index("y")
    p = _matmul(x, dy)
    recv = _comm(p)
    mine = lax.dynamic_slice(p, (my_y * HALF, 0), (HALF, N))
    return _add(mine, recv)


# device time: 382770 ns/iter; 3.2202x vs baseline; 1.5589x over previous
import jax
import jax.numpy as jnp
from jax import lax
from jax.experimental import pallas as pl
from jax.experimental.pallas import tpu as pltpu

M = 4096
N = 8192
K_LOC = 4096
HALF = M // 2
QTR = HALF // 4

MC = 3072

N_BLK = 1024
K_BLK = 512
J_STEPS = N // N_BLK
K_STEPS = K_LOC // K_BLK


def _ring(my_x, my_y, my_z):
    r = 2 * my_x + (my_z ^ my_x)
    opp = (r + 2) % 4
    left_r = (r + 3) % 4
    right_r = (r + 1) % 4
    r_even = (r % 2) == 0
    right_x = jnp.where(r_even, my_x, 1 - my_x)
    right_z = jnp.where(r_even, 1 - my_z, my_z)
    left_x = jnp.where(r_even, 1 - my_x, my_x)
    left_z = jnp.where(r_even, my_z, 1 - my_z)
    right = (right_x, my_y, right_z)
    left = (left_x, my_y, left_z)
    return r, opp, left_r, right_r, left, right


def _prep_body(cols_ref, x_ref, o_ref):
    del cols_ref
    o_ref[...] = x_ref[...].astype(jnp.bfloat16)


def _prep(cols, x):
    grid = (MC // 512, K_LOC // 512)
    return pl.pallas_call(
        _prep_body,
        grid_spec=pltpu.PrefetchScalarGridSpec(
            num_scalar_prefetch=1,
            grid=grid,
            in_specs=[
                pl.BlockSpec((512, 512), lambda c, kb, cols: (kb, cols[c])),
            ],
            out_specs=pl.BlockSpec((512, 512), lambda c, kb, cols: (kb, c)),
        ),
        out_shape=jax.ShapeDtypeStruct((K_LOC, MC), jnp.bfloat16),
    )(cols, x)


def _main_body(x_ref, dy_ref, mine_ref, recv_ref, acc_ref, sendbuf,
               as_s, as_r, fw_s, fw_r):
    j = pl.program_id(0)
    k = pl.program_id(1)
    my_x = lax.axis_index("x")
    my_y = lax.axis_index("y")
    my_z = lax.axis_index("z")
    other = 1 - my_y
    r, opp, left_r, right_r, left, right = _ring(my_x, my_y, my_z)
    y_nbr = (my_x, other, my_z)

    def band(b):
        return pl.ds(b * N_BLK, N_BLK)

    def slot(s, b):
        return recv_ref.at[pl.ds(s * QTR, QTR), band(b)]

    def a_rdma(q, b):
        dst_slot = r if q == 0 else opp
        return pltpu.make_async_remote_copy(
            src_ref=sendbuf.at[q, b],
            dst_ref=slot(dst_slot, b),
            send_sem=as_s.at[q, b],
            recv_sem=as_r.at[q, b],
            device_id=y_nbr,
            device_id_type=pl.DeviceIdType.MESH,
        )

    def fw_rdma(d, b):
        return pltpu.make_async_remote_copy(
            src_ref=slot(r, b),
            dst_ref=slot(r, b),
            send_sem=fw_s.at[d, b],
            recv_sem=fw_r.at[d, b],
            device_id=right if d == 0 else left,
            device_id_type=pl.DeviceIdType.MESH,
        )

    def fw_wait(d, b):
        return pltpu.make_async_remote_copy(
            src_ref=slot(r, b),
            dst_ref=slot(left_r if d == 0 else right_r, b),
            send_sem=fw_s.at[d, b],
            recv_sem=fw_r.at[d, b],
            device_id=left if d == 0 else right,
            device_id_type=pl.DeviceIdType.MESH,
        )

    @pl.when(k == 0)
    def _():
        acc_ref[...] = jnp.zeros_like(acc_ref)

    acc_ref[...] += lax.dot_general(
        x_ref[...], dy_ref[...].astype(jnp.bfloat16),
        (((0,), (0,)), ((), ())),
        preferred_element_type=jnp.float32,
    )

    @pl.when(k == K_STEPS - 1)
    def _():
        mine_ref[...] = acc_ref[2 * QTR:, :].astype(jnp.bfloat16)
        sendbuf[0, j] = acc_ref[0:QTR, :].astype(jnp.bfloat16)
        sendbuf[1, j] = acc_ref[QTR:2 * QTR, :].astype(jnp.bfloat16)
        a0 = a_rdma(0, j)
        a0.start()
        a1 = a_rdma(1, j)
        a1.start()

    @pl.when(jnp.logical_and(j > 0, k == 3))
    def _():
        b = j - 1
        a_rdma(0, b).wait_recv()
        fw_rdma(0, b).start()
        fw_rdma(1, b).start()

    @pl.when(jnp.logical_and(j == J_STEPS - 1, k == K_STEPS - 1))
    def _():
        b = J_STEPS - 1
        a_rdma(0, b).wait_recv()
        fw_rdma(0, b).start()
        fw_rdma(1, b).start()
        for bb in range(J_STEPS):
            a_rdma(1, bb).wait_recv()
            fw_wait(0, bb).wait_recv()
            fw_wait(1, bb).wait_recv()
            a_rdma(0, bb).wait_send()
            a_rdma(1, bb).wait_send()
            fw_rdma(0, bb).wait_send()
            fw_rdma(1, bb).wait_send()


def _main(x_cat, dy):
    grid = (J_STEPS, K_STEPS)
    return pl.pallas_call(
        _main_body,
        grid=grid,
        in_specs=[
            pl.BlockSpec((K_BLK, MC), lambda j, k: (k, 0)),
            pl.BlockSpec((K_BLK, N_BLK), lambda j, k: (k, j)),
        ],
        out_specs=[
            pl.BlockSpec((HALF, N_BLK), lambda j, k: (0, j)),
            pl.BlockSpec(memory_space=pl.ANY),
        ],
        out_shape=[
            jax.ShapeDtypeStruct((HALF, N), jnp.bfloat16),
            jax.ShapeDtypeStruct((HALF, N), jnp.bfloat16),
        ],
        scratch_shapes=[
            pltpu.VMEM((MC, N_BLK), jnp.float32),
            pltpu.VMEM((2, J_STEPS, QTR, N_BLK), jnp.bfloat16),
            pltpu.SemaphoreType.DMA((2, J_STEPS)),
            pltpu.SemaphoreType.DMA((2, J_STEPS)),
            pltpu.SemaphoreType.DMA((2, J_STEPS)),
            pltpu.SemaphoreType.DMA((2, J_STEPS)),
        ],
        compiler_params=pltpu.CompilerParams(
            dimension_semantics=("arbitrary", "arbitrary"),
            vmem_limit_bytes=100 * 1024 * 1024,
        ),
    )(x_cat, dy)


def _add_body(a_ref, b_ref, o_ref):
    o_ref[...] = a_ref[...].astype(jnp.float32) + b_ref[...].astype(jnp.float32)


_ADD_BLK = 128


def _add(a, b):
    grid = (HALF // _ADD_BLK,)
    spec = pl.BlockSpec((_ADD_BLK, N), lambda i: (i, 0))
    return pl.pallas_call(
        _add_body,
        grid=grid,
        in_specs=[spec, spec],
        out_specs=pl.BlockSpec((_ADD_BLK, N), lambda i: (i, 0)),
        out_shape=jax.ShapeDtypeStruct((HALF, N), jnp.float32),
    )(a, b)


def kernel(x, dy):
    my_x = lax.axis_index("x")
    my_y = lax.axis_index("y")
    my_z = lax.axis_index("z")
    other = 1 - my_y
    r, opp, _, _, _, _ = _ring(my_x, my_y, my_z)
    cols = jnp.stack([
        other * 4 + r,
        other * 4 + opp,
        my_y * 4 + 0,
        my_y * 4 + 1,
        my_y * 4 + 2,
        my_y * 4 + 3,
    ]).astype(jnp.int32)
    x_cat = _prep(cols, x)
    mine, recv = _main(x_cat, dy)
    return _add(mine, recv)
